# baseline (speedup 1.0000x reference)
import functools

import jax
import jax.numpy as jnp
from jax import lax
from jax.experimental import pallas as pl
from jax.experimental.pallas import tpu as pltpu
from jax.experimental.pallas import tpu_sc as plsc

B = 32
N_VERTS = 100000
N_FACES = 200000
NC, NS, L = 2, 16, 16
CHUNK = 128
NCHUNK = 98
FACES_PAD = NS * NCHUNK * CHUNK
D = B * 3
NCOL = D // L


def _sc_body(vt_hbm, idx_hbm, out_hbm, idx_v, buf_v, out_stage, sem0, sem1):
    cid = lax.axis_index("c")
    sid = lax.axis_index("s")
    wid = sid * NC + cid
    sems = (sem0, sem1)

    @pl.when(cid == 0)
    def _work():
        pltpu.sync_copy(idx_hbm.at[sid], idx_v)

        def fire(k, slot):
            for g in range(3):
                pltpu.async_copy(vt_hbm.at[idx_v.at[g, k]],
                                 buf_v.at[slot, g], sems[slot])

        def drain(k, slot):
            for g in range(3):
                pltpu.make_async_copy(vt_hbm.at[idx_v.at[g, k]],
                                      buf_v.at[slot, g], sems[slot]).wait()

        def compute(slot, accs):
            def unpack2(u):
                ev = lax.bitcast_convert_type(u << 16, jnp.float32)
                od = lax.bitcast_convert_type(u, jnp.float32)
                return ev, od

            def face_body(j, accs):
                accs = list(accs)
                for c in range(NCOL // 2):
                    a = buf_v[slot, 0, j, pl.ds(L * c, L)]
                    b = buf_v[slot, 1, j, pl.ds(L * c, L)]
                    d = buf_v[slot, 2, j, pl.ds(L * c, L)]
                    for h, (ae, be, de) in enumerate(
                            zip(unpack2(a), unpack2(b), unpack2(d))):
                        hi = jnp.maximum(ae, jnp.maximum(be, de))
                        lo = jnp.minimum(ae, jnp.minimum(be, de))
                        cc = 2 * c + h
                        accs[cc] = accs[cc] + (hi - lo)
                return tuple(accs)
            return lax.fori_loop(0, CHUNK, face_body, accs)

        fire(0, 0)
        zero = jnp.zeros((L,), jnp.float32)
        accs = (zero,) * NCOL

        def chunk_step(k, slot, accs):
            @pl.when(k + 1 < NCHUNK)
            def _():
                fire(k + 1, 1 - slot)
            drain(k, slot)
            return compute(slot, accs)

        def pair_body(i, accs):
            accs = chunk_step(2 * i, 0, accs)
            accs = chunk_step(2 * i + 1, 1, accs)
            return accs

        accs = lax.fori_loop(0, NCHUNK // 2, pair_body, accs)

        total = accs[0]
        for c in range(1, NCOL):
            total = total + accs[c]
        out_stage[...] = total
        pltpu.sync_copy(out_stage, out_hbm.at[wid])

    @pl.when(cid != 0)
    def _idle():
        out_stage[...] = jnp.zeros((L,), jnp.float32)
        pltpu.sync_copy(out_stage, out_hbm.at[wid])


@functools.partial(
    pl.kernel,
    out_type=jax.ShapeDtypeStruct((NC * NS, L), jnp.float32),
    mesh=plsc.VectorSubcoreMesh(core_axis_name="c", subcore_axis_name="s"),
    compiler_params=pltpu.CompilerParams(use_tc_tiling_on_sc=False),
    scratch_types=[
        pltpu.VMEM((3, NCHUNK, CHUNK), jnp.int32),
        pltpu.VMEM((2, 3, CHUNK, D // 2), jnp.int32),
        pltpu.VMEM((L,), jnp.float32),
        pltpu.SemaphoreType.DMA,
        pltpu.SemaphoreType.DMA,
    ],
)
def _edge_loss_sc(vt_hbm, idx_hbm, out_hbm, idx_v, buf_v, out_stage,
                  sem0, sem1):
    _sc_body(vt_hbm, idx_hbm, out_hbm, idx_v, buf_v, out_stage, sem0, sem1)


def kernel(v, faces):
    vt = jnp.transpose(v, (1, 0, 2)).reshape(N_VERTS, D // 2, 2)
    vt = lax.bitcast_convert_type(vt.astype(jnp.bfloat16), jnp.int32)
    fi = faces.astype(jnp.int32).T
    fi = jnp.pad(fi, ((0, 0), (0, FACES_PAD - N_FACES)))
    fi = fi.reshape(3, NS, NCHUNK, CHUNK).transpose(1, 0, 2, 3)
    partials = _edge_loss_sc(vt, fi)
    return 2.0 * jnp.sum(partials) / jnp.float32(B * N_FACES * 3)


# speedup vs baseline: 1.0693x; 1.0693x over previous
import functools

import jax
import jax.numpy as jnp
from jax import lax
from jax.experimental import pallas as pl
from jax.experimental.pallas import tpu as pltpu
from jax.experimental.pallas import tpu_sc as plsc

B = 32
N_VERTS = 100000
N_FACES = 200000
NC, NS, L = 2, 16, 16
CHUNK = 128
NCHUNK = 98
FACES_PAD = NS * NCHUNK * CHUNK
D = B * 3
NCOL = D // L


def _sc_body(vt_hbm, idx_hbm, out_hbm, idx_v, buf_v, out_stage, sem0, sem1):
    cid = lax.axis_index("c")
    sid = lax.axis_index("s")
    wid = sid * NC + cid
    sems = (sem0, sem1)

    @pl.when(cid == 0)
    def _work():
        pltpu.sync_copy(idx_hbm.at[sid], idx_v)

        def fire(k, slot):
            for g in range(3):
                pltpu.async_copy(vt_hbm.at[idx_v.at[g, k]],
                                 buf_v.at[slot, g], sems[slot])

        def drain(k, slot):
            for g in range(3):
                pltpu.make_async_copy(vt_hbm.at[idx_v.at[g, k]],
                                      buf_v.at[slot, g], sems[slot]).wait()

        def compute(slot, accs):
            def unpack2(u):
                ev = lax.bitcast_convert_type(u << 16, jnp.float32)
                od = lax.bitcast_convert_type(u, jnp.float32)
                return ev, od

            def face_body(j, accs):
                accs = list(accs)
                for c in range(NCOL // 2):
                    a = buf_v[slot, 0, j, pl.ds(L * c, L)]
                    b = buf_v[slot, 1, j, pl.ds(L * c, L)]
                    d = buf_v[slot, 2, j, pl.ds(L * c, L)]
                    for h, (ae, be, de) in enumerate(
                            zip(unpack2(a), unpack2(b), unpack2(d))):
                        hi = jnp.maximum(ae, jnp.maximum(be, de))
                        lo = jnp.minimum(ae, jnp.minimum(be, de))
                        cc = 2 * c + h
                        accs[cc] = accs[cc] + (hi - lo)
                return tuple(accs)
            return lax.fori_loop(0, CHUNK, face_body, accs)

        fire(0, 0)
        zero = jnp.zeros((L,), jnp.float32)
        accs = (zero,) * NCOL

        def chunk_step(k, slot, accs):
            @pl.when(k + 1 < NCHUNK)
            def _():
                fire(k + 1, 1 - slot)
            drain(k, slot)
            return compute(slot, accs)

        def pair_body(i, accs):
            accs = chunk_step(2 * i, 0, accs)
            accs = chunk_step(2 * i + 1, 1, accs)
            return accs

        accs = lax.fori_loop(0, NCHUNK // 2, pair_body, accs)

        total = accs[0]
        for c in range(1, NCOL):
            total = total + accs[c]
        out_stage[...] = total
        pltpu.sync_copy(out_stage, out_hbm.at[wid])

    @pl.when(cid != 0)
    def _idle():
        out_stage[...] = jnp.zeros((L,), jnp.float32)
        pltpu.sync_copy(out_stage, out_hbm.at[wid])


@functools.partial(
    pl.kernel,
    out_type=jax.ShapeDtypeStruct((NC * NS, L), jnp.float32),
    mesh=plsc.VectorSubcoreMesh(core_axis_name="c", subcore_axis_name="s"),
    compiler_params=pltpu.CompilerParams(use_tc_tiling_on_sc=False),
    scratch_types=[
        pltpu.VMEM((3, NCHUNK, CHUNK), jnp.int32),
        pltpu.VMEM((2, 3, CHUNK, D // 2), jnp.int32),
        pltpu.VMEM((L,), jnp.float32),
        pltpu.SemaphoreType.DMA,
        pltpu.SemaphoreType.DMA,
    ],
)
def _edge_loss_sc(vt_hbm, idx_hbm, out_hbm, idx_v, buf_v, out_stage,
                  sem0, sem1):
    _sc_body(vt_hbm, idx_hbm, out_hbm, idx_v, buf_v, out_stage, sem0, sem1)


def kernel(v, faces):
    ui = lax.bitcast_convert_type(v, jnp.uint32)
    rnd = (ui + jnp.uint32(0x7FFF) + ((ui >> 16) & jnp.uint32(1))) >> 16
    t = rnd.transpose(1, 0, 2).reshape(N_VERTS, D // 2, 2)
    vt = lax.bitcast_convert_type(
        (t[..., 0] | (t[..., 1] << 16)), jnp.int32)
    fi = faces.astype(jnp.int32).T
    fi = jnp.pad(fi, ((0, 0), (0, FACES_PAD - N_FACES)))
    fi = fi.reshape(3, NS, NCHUNK, CHUNK).transpose(1, 0, 2, 3)
    partials = _edge_loss_sc(vt, fi)
    return 2.0 * jnp.sum(partials) / jnp.float32(B * N_FACES * 3)


# speedup vs baseline: 1.3450x; 1.2579x over previous
import functools

import jax
import jax.numpy as jnp
from jax import lax
from jax.experimental import pallas as pl
from jax.experimental.pallas import tpu as pltpu
from jax.experimental.pallas import tpu_sc as plsc

B = 32
N_VERTS = 100000
N_FACES = 200000
NC, NS, L = 2, 16, 16
CHUNK = 128
NCHUNK = 98
FACES_PAD = NS * NCHUNK * CHUNK
D = B * 3
NCOL = D // L


def _sc_body(vt_hbm, idx_hbm, out_hbm, idx_v, buf_v, out_stage, sem0, sem1):
    cid = lax.axis_index("c")
    sid = lax.axis_index("s")
    wid = sid * NC + cid
    sems = (sem0, sem1)

    @pl.when(cid == 0)
    def _work():
        pltpu.sync_copy(idx_hbm.at[sid], idx_v)

        def fire(k, slot):
            for q in range(3):
                pltpu.async_copy(
                    vt_hbm.at[idx_v.at[pl.ds(k * 3 * CHUNK + q * CHUNK,
                                             CHUNK)]],
                    buf_v.at[slot, pl.ds(q * CHUNK, CHUNK)], sems[slot])

        def drain(k, slot):
            for q in range(3):
                pltpu.make_async_copy(
                    vt_hbm.at[idx_v.at[pl.ds(k * 3 * CHUNK + q * CHUNK,
                                             CHUNK)]],
                    buf_v.at[slot, pl.ds(q * CHUNK, CHUNK)],
                    sems[slot]).wait()

        def compute(slot, accs):
            def unpack2(u):
                ev = lax.bitcast_convert_type(u << 16, jnp.float32)
                od = lax.bitcast_convert_type(u, jnp.float32)
                return ev, od

            def face_body(j, accs):
                accs = list(accs)
                r = 3 * j
                for c in range(NCOL // 2):
                    a = buf_v[slot, r, pl.ds(L * c, L)]
                    b = buf_v[slot, r + 1, pl.ds(L * c, L)]
                    d = buf_v[slot, r + 2, pl.ds(L * c, L)]
                    for h, (ae, be, de) in enumerate(
                            zip(unpack2(a), unpack2(b), unpack2(d))):
                        hi = jnp.maximum(ae, jnp.maximum(be, de))
                        lo = jnp.minimum(ae, jnp.minimum(be, de))
                        cc = 2 * c + h
                        accs[cc] = accs[cc] + (hi - lo)
                return tuple(accs)
            return lax.fori_loop(0, CHUNK, face_body, accs)

        fire(0, 0)
        zero = jnp.zeros((L,), jnp.float32)
        accs = (zero,) * NCOL

        def chunk_step(k, slot, accs):
            @pl.when(k + 1 < NCHUNK)
            def _():
                fire(k + 1, 1 - slot)
            drain(k, slot)
            return compute(slot, accs)

        def pair_body(i, accs):
            accs = chunk_step(2 * i, 0, accs)
            accs = chunk_step(2 * i + 1, 1, accs)
            return accs

        accs = lax.fori_loop(0, NCHUNK // 2, pair_body, accs)

        total = accs[0]
        for c in range(1, NCOL):
            total = total + accs[c]
        out_stage[...] = total
        pltpu.sync_copy(out_stage, out_hbm.at[wid])

    @pl.when(cid != 0)
    def _idle():
        out_stage[...] = jnp.zeros((L,), jnp.float32)
        pltpu.sync_copy(out_stage, out_hbm.at[wid])


@functools.partial(
    pl.kernel,
    out_type=jax.ShapeDtypeStruct((NC * NS, L), jnp.float32),
    mesh=plsc.VectorSubcoreMesh(core_axis_name="c", subcore_axis_name="s"),
    compiler_params=pltpu.CompilerParams(use_tc_tiling_on_sc=False),
    scratch_types=[
        pltpu.VMEM((NCHUNK * 3 * CHUNK,), jnp.int32),
        pltpu.VMEM((2, 3 * CHUNK, D // 2), jnp.int32),
        pltpu.VMEM((L,), jnp.float32),
        pltpu.SemaphoreType.DMA,
        pltpu.SemaphoreType.DMA,
    ],
)
def _edge_loss_sc(vt_hbm, idx_hbm, out_hbm, idx_v, buf_v, out_stage,
                  sem0, sem1):
    _sc_body(vt_hbm, idx_hbm, out_hbm, idx_v, buf_v, out_stage, sem0, sem1)


def kernel(v, faces):
    ui = lax.bitcast_convert_type(v, jnp.uint32)
    rnd = (ui + jnp.uint32(0x7FFF) + ((ui >> 16) & jnp.uint32(1))) >> 16
    packed = rnd[: B // 2] | (rnd[B // 2:] << 16)
    vt = lax.bitcast_convert_type(
        packed.transpose(1, 0, 2).reshape(N_VERTS, D // 2), jnp.int32)
    fi = jnp.pad(faces.astype(jnp.int32).reshape(-1),
                 (0, 3 * (FACES_PAD - N_FACES)))
    fi = fi.reshape(NS, NCHUNK * 3 * CHUNK)
    partials = _edge_loss_sc(vt, fi)
    return 2.0 * jnp.sum(partials) / jnp.float32(B * N_FACES * 3)
